# Initial kernel scaffold; baseline (speedup 1.0000x reference)
import jax
import jax.numpy as jnp
from jax.experimental import pallas as pl


def kernel(x, calendar_features, W_feat, b_feat, pos_table, day_table, month_table):
    raise NotImplementedError("write your pallas kernel here")


# speedup vs baseline: 5.9484x; 5.9484x over previous
import jax
import jax.numpy as jnp
from jax.experimental import pallas as pl

_RB = 32


def _fused_body(x_ref, di_ref, mi_ref, w_ref, tab_ref, pos_ref, b_ref, o_ref):
    acc = jnp.dot(x_ref[...], w_ref[...], preferred_element_type=jnp.float32)
    iota = jax.lax.broadcasted_iota(jnp.int32, (1, 24), 1)
    onehot = ((di_ref[...] == iota) | ((mi_ref[...] + 8) == iota)).astype(
        jnp.float32
    )
    acc = acc + jnp.dot(onehot, tab_ref[...], preferred_element_type=jnp.float32)
    o_ref[...] = acc + pos_ref[...] + b_ref[...]


@jax.jit
def kernel(x, calendar_features, W_feat, b_feat, pos_table, day_table, month_table):
    B, L, F = x.shape
    D = W_feat.shape[1]
    N = B * L
    R = _RB * L
    x2 = x.reshape(N, F)
    cal = calendar_features.astype(jnp.int32)
    day_col = cal[:, :, 0].reshape(N, 1)
    month_col = cal[:, :, 1].reshape(N, 1)
    tab = jnp.zeros((24, D), dtype=jnp.float32)
    tab = tab.at[0:7].set(day_table)
    tab = tab.at[8:20].set(month_table)
    pos_rep = jnp.tile(pos_table[:L], (_RB, 1))
    bvec = b_feat.reshape(1, D)

    out2 = pl.pallas_call(
        _fused_body,
        grid=(N // R,),
        in_specs=[
            pl.BlockSpec((R, F), lambda i: (i, 0)),
            pl.BlockSpec((R, 1), lambda i: (i, 0)),
            pl.BlockSpec((R, 1), lambda i: (i, 0)),
            pl.BlockSpec((F, D), lambda i: (0, 0)),
            pl.BlockSpec((24, D), lambda i: (0, 0)),
            pl.BlockSpec((R, D), lambda i: (0, 0)),
            pl.BlockSpec((1, D), lambda i: (0, 0)),
        ],
        out_specs=pl.BlockSpec((R, D), lambda i: (i, 0)),
        out_shape=jax.ShapeDtypeStruct((N, D), jnp.float32),
    )(x2, day_col, month_col, W_feat, tab, pos_rep, bvec)
    return out2.reshape(B, L, D)


# speedup vs baseline: 9.8554x; 1.6568x over previous
import jax
import jax.numpy as jnp
from jax.experimental import pallas as pl

_RB = 32


def _fused_body(x_ref, code_ref, w_ref, tab_ref, pos_ref, b_ref, o_ref):
    acc = jnp.dot(x_ref[...], w_ref[...], preferred_element_type=jnp.float32)
    iota = jax.lax.broadcasted_iota(jnp.int32, (1, 24), 1)
    code = code_ref[...].astype(jnp.int32)
    di = jnp.bitwise_and(code, 7)
    mi = jnp.right_shift(code, 3)
    onehot = ((di == iota) | ((mi + 8) == iota)).astype(jnp.float32)
    acc = acc + jnp.dot(onehot, tab_ref[...], preferred_element_type=jnp.float32)
    o_ref[...] = acc + pos_ref[...] + b_ref[...]


@jax.jit
def kernel(x, calendar_features, W_feat, b_feat, pos_table, day_table, month_table):
    B, L, F = x.shape
    D = W_feat.shape[1]
    N = B * L
    R = _RB * L
    x2 = x.reshape(N, F)
    cal = calendar_features.astype(jnp.int32)
    code_col = (cal[:, :, 0] | (cal[:, :, 1] << 3)).astype(jnp.int8).reshape(N, 1)
    tab = jnp.zeros((24, D), dtype=jnp.float32)
    tab = tab.at[0:7].set(day_table)
    tab = tab.at[8:20].set(month_table)
    pos_rep = jnp.tile(pos_table[:L], (_RB, 1))
    bvec = b_feat.reshape(1, D)

    out2 = pl.pallas_call(
        _fused_body,
        grid=(N // R,),
        in_specs=[
            pl.BlockSpec((R, F), lambda i: (i, 0)),
            pl.BlockSpec((R, 1), lambda i: (i, 0)),
            pl.BlockSpec((F, D), lambda i: (0, 0)),
            pl.BlockSpec((24, D), lambda i: (0, 0)),
            pl.BlockSpec((R, D), lambda i: (0, 0)),
            pl.BlockSpec((1, D), lambda i: (0, 0)),
        ],
        out_specs=pl.BlockSpec((R, D), lambda i: (i, 0)),
        out_shape=jax.ShapeDtypeStruct((N, D), jnp.float32),
    )(x2, code_col, W_feat, tab, pos_rep, bvec)
    return out2.reshape(B, L, D)


# speedup vs baseline: 12.0330x; 1.2210x over previous
import jax
import jax.numpy as jnp
from jax.experimental import pallas as pl

_RB = 32


def _fused_body(x_ref, code_ref, w_ref, tab_ref, pos_ref, b_ref, o_ref):
    acc = jnp.dot(x_ref[...], w_ref[...], preferred_element_type=jnp.float32)
    code = code_ref[0]
    iota = jax.lax.broadcasted_iota(jnp.int32, (24, 1), 0)
    di = jnp.bitwise_and(code, 7)
    mi = jnp.right_shift(code, 3)
    onehot_t = ((di == iota) | ((mi + 8) == iota)).astype(jnp.float32)
    cal_emb = jax.lax.dot_general(
        onehot_t,
        tab_ref[...],
        dimension_numbers=(((0,), (0,)), ((), ())),
        preferred_element_type=jnp.float32,
    )
    o_ref[...] = acc + cal_emb + pos_ref[...] + b_ref[...]


@jax.jit
def kernel(x, calendar_features, W_feat, b_feat, pos_table, day_table, month_table):
    B, L, F = x.shape
    D = W_feat.shape[1]
    N = B * L
    R = _RB * L
    NB = N // R
    x2 = x.reshape(N, F)
    cal = calendar_features.astype(jnp.int32)
    code3 = (cal[:, :, 0] | (cal[:, :, 1] << 3)).reshape(NB, 1, R)
    tab = jnp.zeros((24, D), dtype=jnp.float32)
    tab = tab.at[0:7].set(day_table)
    tab = tab.at[8:20].set(month_table)
    pos_rep = jnp.tile(pos_table[:L], (_RB, 1))
    bvec = b_feat.reshape(1, D)

    out2 = pl.pallas_call(
        _fused_body,
        grid=(NB,),
        in_specs=[
            pl.BlockSpec((R, F), lambda i: (i, 0)),
            pl.BlockSpec((1, 1, R), lambda i: (i, 0, 0)),
            pl.BlockSpec((F, D), lambda i: (0, 0)),
            pl.BlockSpec((24, D), lambda i: (0, 0)),
            pl.BlockSpec((R, D), lambda i: (0, 0)),
            pl.BlockSpec((1, D), lambda i: (0, 0)),
        ],
        out_specs=pl.BlockSpec((R, D), lambda i: (i, 0)),
        out_shape=jax.ShapeDtypeStruct((N, D), jnp.float32),
    )(x2, code3, W_feat, tab, pos_rep, bvec)
    return out2.reshape(B, L, D)
